# baseline (speedup 1.0000x reference)
import dataclasses
import functools

import jax
import jax.numpy as jnp
from jax import lax
from jax.experimental import pallas as pl
from jax.experimental.pallas import tpu as pltpu
from jax.experimental.pallas import tpu_sc as plsc

B, C, H, W = 8, 3, 512, 512
N = 256
FP = 16
ED = 768
P = B * N
K = C * FP
D = C * FP * FP
L = 16
NC, NS = 2, 16
NWORK = NC * NS
PPW = P // NWORK
HALF = PPW // 2
CHO = 8
NGH = HALF // CHO
NROWS = B * C * H * W // L


def _sc_gather(img_rows, gidx, shifts):
    mesh = plsc.VectorSubcoreMesh(core_axis_name="c", subcore_axis_name="s")
    cp = pltpu.CompilerParams()
    if "needs_layout_passes" in pltpu.CompilerParams.__dataclass_fields__:
        cp = dataclasses.replace(cp, needs_layout_passes=False)
    if "use_tc_tiling_on_sc" in pltpu.CompilerParams.__dataclass_fields__:
        cp = dataclasses.replace(cp, use_tc_tiling_on_sc=False)

    @functools.partial(
        pl.kernel,
        out_type=jax.ShapeDtypeStruct((P * (D // 128), 128), jnp.float32),
        mesh=mesh,
        compiler_params=cp,
        scratch_types=[
            pltpu.VMEM((PPW, 2 * K), jnp.int32),
            pltpu.VMEM((PPW, L), jnp.int32),
            pltpu.VMEM((PPW, 2 * K, L), jnp.float32),
            pltpu.VMEM((2, CHO * (D // 128), 128), jnp.float32),
            pltpu.SemaphoreType.DMA,
            pltpu.SemaphoreType.DMA,
            pltpu.SemaphoreType.DMA,
        ],
    )
    def k(img_hbm, gidx_hbm, shf_hbm, out_hbm,
          idx_v, shf_v, win_v, out_v, gsem_a, gsem_b, osem):
        wid = lax.axis_index("c") * NS + lax.axis_index("s")
        wbase = wid * PPW
        lanes = lax.iota(jnp.int32, L)

        pltpu.sync_copy(gidx_hbm.at[pl.ds(wbase, PPW)], idx_v)
        pltpu.sync_copy(shf_hbm.at[pl.ds(wbase, PPW)], shf_v)

        @pl.loop(0, HALF)
        def _fire_a(p):
            pltpu.async_copy(img_hbm.at[idx_v.at[p]], win_v.at[p], gsem_a)

        @pl.loop(HALF, PPW)
        def _fire_b(p):
            pltpu.async_copy(img_hbm.at[idx_v.at[p]], win_v.at[p], gsem_b)

        def extract_half(base, goff, gsem):
            @pl.loop(0, HALF)
            def _drain(p):
                pltpu.make_async_copy(
                    img_hbm.at[idx_v.at[p]], win_v.at[p], gsem).wait()

            @pl.loop(0, NGH)
            def _group(g):
                gg = goff + g
                buf = gg & 1

                @pl.when(gg >= 2)
                def _():
                    pltpu.make_async_copy(
                        out_v.at[buf],
                        out_hbm.at[pl.ds(0, CHO * (D // 128))], osem).wait()

                @pl.loop(0, CHO)
                def _patch(jj):
                    p = base + g * CHO + jj
                    t = shf_v[p] + lanes
                    ib0 = t >> 4
                    i1 = t & 15
                    win2 = win_v.at[p]

                    @pl.loop(0, K, unroll=8)
                    def _row(kk):
                        v = plsc.load_gather(win2, [ib0 + kk * 2, i1])
                        out_v[buf, jj * (D // 128) + (kk >> 3),
                              pl.ds((kk & 7) * 16, L)] = v

                pltpu.async_copy(
                    out_v.at[buf],
                    out_hbm.at[pl.ds((wbase + base + g * CHO) * (D // 128),
                                     CHO * (D // 128))], osem)

        extract_half(0, 0, gsem_a)
        extract_half(HALF, NGH, gsem_b)

        @pl.loop(0, 2)
        def _final(i):
            pltpu.make_async_copy(
                out_v.at[i], out_hbm.at[pl.ds(0, CHO * (D // 128))],
                osem).wait()

    return k(img_rows, gidx, shifts)


def _tc_matmul(flat2, w, bias):
    rb = 256
    rows = D // 128

    def mm(x_ref, w_ref, b_ref, o_ref):
        xb = x_ref[...].reshape(rb, D)
        o_ref[...] = lax.dot_general(
            xb, w_ref[...], (((1,), (1,)), ((), ())),
            preferred_element_type=jnp.float32) + b_ref[...]

    return pl.pallas_call(
        mm,
        grid=(P // rb,),
        in_specs=[
            pl.BlockSpec((rb * rows, 128), lambda i: (i, 0)),
            pl.BlockSpec((ED, D), lambda i: (0, 0)),
            pl.BlockSpec((1, ED), lambda i: (0, 0)),
        ],
        out_specs=pl.BlockSpec((rb, ED), lambda i: (i, 0)),
        out_shape=jax.ShapeDtypeStruct((P, ED), jnp.float32),
    )(flat2, w, bias.reshape(1, ED))


def kernel(images, patch_locations, W_fine, b_fine, W_coarse, b_coarse):
    del W_coarse, b_coarse
    locs = patch_locations.reshape(P, 5)
    x = locs[:, 0]
    y = locs[:, 1]
    bidx = jnp.repeat(jnp.arange(B, dtype=jnp.int32), N)
    c = jnp.arange(C, dtype=jnp.int32)
    i = jnp.arange(FP, dtype=jnp.int32)
    e = ((bidx[:, None, None] * C + c[None, :, None]) * (H * W)
         + (y[:, None, None] + i[None, None, :]) * W + x[:, None, None])
    r0 = e >> 4
    r1 = jnp.minimum(r0 + 1, NROWS - 1)
    gidx = jnp.stack([r0, r1], axis=-1).reshape(P, 2 * K)
    shifts = jnp.broadcast_to((x & 15)[:, None], (P, L)).astype(jnp.int32)
    img_rows = lax.optimization_barrier(
        images.reshape(NROWS * L // 128, 128)).reshape(NROWS, L)

    flat = _sc_gather(img_rows, gidx, shifts)
    emb = _tc_matmul(flat, W_fine, b_fine)
    return emb.reshape(B, N, ED)


# speedup vs baseline: 1.0800x; 1.0800x over previous
import dataclasses
import functools

import jax
import jax.numpy as jnp
from jax import lax
from jax.experimental import pallas as pl
from jax.experimental.pallas import tpu as pltpu
from jax.experimental.pallas import tpu_sc as plsc

B, C, H, W = 8, 3, 512, 512
N = 256
FP = 16
ED = 768
P = B * N
K = C * FP
D = C * FP * FP
L = 16
NC, NS = 2, 16
NWORK = NC * NS
PPW = P // NWORK
HALF = PPW // 2
CHO = 8
NGH = HALF // CHO
NROWS = B * C * H * W // L


def _sc_gather(img_rows, gidx, shifts):
    mesh = plsc.VectorSubcoreMesh(core_axis_name="c", subcore_axis_name="s")
    cp = pltpu.CompilerParams()
    if "needs_layout_passes" in pltpu.CompilerParams.__dataclass_fields__:
        cp = dataclasses.replace(cp, needs_layout_passes=False)
    if "use_tc_tiling_on_sc" in pltpu.CompilerParams.__dataclass_fields__:
        cp = dataclasses.replace(cp, use_tc_tiling_on_sc=False)

    @functools.partial(
        pl.kernel,
        out_type=jax.ShapeDtypeStruct((P * (D // 128), 128), jnp.float32),
        mesh=mesh,
        compiler_params=cp,
        scratch_types=[
            pltpu.VMEM((PPW, 2 * K), jnp.int32),
            pltpu.VMEM((PPW, L), jnp.int32),
            pltpu.VMEM((PPW, 2 * K, L), jnp.float32),
            pltpu.VMEM((2, CHO * (D // 128), 128), jnp.float32),
            pltpu.SemaphoreType.DMA,
            pltpu.SemaphoreType.DMA,
            pltpu.SemaphoreType.DMA,
        ],
    )
    def k(img_hbm, gidx_hbm, shf_hbm, out_hbm,
          idx_v, shf_v, win_v, out_v, gsem_a, gsem_b, osem):
        wid = lax.axis_index("c") * NS + lax.axis_index("s")
        wbase = wid * PPW
        lanes = lax.iota(jnp.int32, L)

        pltpu.sync_copy(gidx_hbm.at[pl.ds(wbase, PPW)], idx_v)
        pltpu.sync_copy(shf_hbm.at[pl.ds(wbase, PPW)], shf_v)

        @pl.loop(0, HALF)
        def _fire_a(p):
            pltpu.async_copy(img_hbm.at[idx_v.at[p]], win_v.at[p], gsem_a)

        @pl.loop(HALF, PPW)
        def _fire_b(p):
            pltpu.async_copy(img_hbm.at[idx_v.at[p]], win_v.at[p], gsem_b)

        def extract_half(base, goff, gsem):
            @pl.loop(0, HALF)
            def _drain(p):
                pltpu.make_async_copy(
                    img_hbm.at[idx_v.at[p]], win_v.at[p], gsem).wait()

            @pl.loop(0, NGH)
            def _group(g):
                gg = goff + g
                buf = gg & 1

                @pl.when(gg >= 2)
                def _():
                    pltpu.make_async_copy(
                        out_v.at[buf],
                        out_hbm.at[pl.ds(0, CHO * (D // 128))], osem).wait()

                @pl.loop(0, CHO)
                def _patch(jj):
                    p = base + g * CHO + jj
                    t = shf_v[p] + lanes
                    ib0 = t >> 4
                    i1 = t & 15
                    win2 = win_v.at[p]

                    @pl.loop(0, K, unroll=8)
                    def _row(kk):
                        v = plsc.load_gather(win2, [ib0 + kk * 2, i1])
                        out_v[buf, jj * (D // 128) + (kk >> 3),
                              pl.ds((kk & 7) * 16, L)] = v

                pltpu.async_copy(
                    out_v.at[buf],
                    out_hbm.at[pl.ds((wbase + base + g * CHO) * (D // 128),
                                     CHO * (D // 128))], osem)

        extract_half(0, 0, gsem_a)
        extract_half(HALF, NGH, gsem_b)

        @pl.loop(0, 2)
        def _final(i):
            pltpu.make_async_copy(
                out_v.at[i], out_hbm.at[pl.ds(0, CHO * (D // 128))],
                osem).wait()

    return k(img_rows, gidx, shifts)


def _tc_matmul(flat2, w, bias):
    rb = 256
    rows = D // 128

    def mm(x_ref, w_ref, b_ref, o_ref):
        xb = x_ref[...].reshape(rb, D).astype(jnp.bfloat16)
        o_ref[...] = lax.dot_general(
            xb, w_ref[...], (((1,), (1,)), ((), ())),
            preferred_element_type=jnp.float32) + b_ref[...]

    return pl.pallas_call(
        mm,
        grid=(P // rb,),
        in_specs=[
            pl.BlockSpec((rb * rows, 128), lambda i: (i, 0)),
            pl.BlockSpec((ED, D), lambda i: (0, 0)),
            pl.BlockSpec((1, ED), lambda i: (0, 0)),
        ],
        out_specs=pl.BlockSpec((rb, ED), lambda i: (i, 0)),
        out_shape=jax.ShapeDtypeStruct((P, ED), jnp.float32),
    )(flat2, w.astype(jnp.bfloat16), bias.reshape(1, ED))


def kernel(images, patch_locations, W_fine, b_fine, W_coarse, b_coarse):
    del W_coarse, b_coarse
    locs = patch_locations.reshape(P, 5)
    x = locs[:, 0]
    y = locs[:, 1]
    bidx = jnp.repeat(jnp.arange(B, dtype=jnp.int32), N)
    c = jnp.arange(C, dtype=jnp.int32)
    i = jnp.arange(FP, dtype=jnp.int32)
    e = ((bidx[:, None, None] * C + c[None, :, None]) * (H * W)
         + (y[:, None, None] + i[None, None, :]) * W + x[:, None, None])
    r0 = e >> 4
    r1 = jnp.minimum(r0 + 1, NROWS - 1)
    gidx = jnp.stack([r0, r1], axis=-1).reshape(P, 2 * K)
    shifts = jnp.broadcast_to((x & 15)[:, None], (P, L)).astype(jnp.int32)
    img_rows = images.reshape(NROWS, L)

    flat = _sc_gather(img_rows, gidx, shifts)
    emb = _tc_matmul(flat, W_fine, b_fine)
    return emb.reshape(B, N, ED)
